# baseline (speedup 1.0000x reference)
import jax
import jax.numpy as jnp
from jax.experimental import pallas as pl
from jax.experimental.pallas import tpu as pltpu

C = 128
I = 64
EPS = 1e-5
R = 16384
BF = jnp.bfloat16
F8 = jnp.float8_e4m3fn
F32 = jnp.float32


def _gather_rows(oh, ab):
    return jax.lax.dot_general(
        oh, ab.astype(BF), (((0,), (0,)), ((), ())),
        preferred_element_type=F32)


def _seg_sum(oh8, vals8):
    return jax.lax.dot_general(
        oh8, vals8, (((1,), (0,)), ((), ())),
        preferred_element_type=F32)


def _outer(f_hbm, seg_ref, W1_ref, b1_ref, T_ref, out_hbm,
           xs_ref, ss_ref, cnt_ref, s2_ref):
    nb = seg_ref.shape[0]

    def onehot_mask(b):
        seg = seg_ref[b][0, :]
        iota = jax.lax.broadcasted_iota(jnp.int32, (I, R), 0)
        return iota == seg[None, :]

    ss_ref[...] = jnp.zeros_like(ss_ref)
    cnt_ref[...] = jnp.zeros_like(cnt_ref)
    s2_ref[...] = jnp.zeros_like(s2_ref)

    W1b = W1_ref[...].astype(BF)
    b1v = b1_ref[...]

    b1b = b1v.astype(BF)

    def l0(idx, f_blk):
        b = idx[0]
        x = jnp.dot(f_blk[...].astype(BF), W1b,
                    preferred_element_type=F32)
        xb = x.astype(BF) + b1b
        xs_ref[pl.ds(b * R, R), :] = xb
        m = onehot_mask(b)
        oh8 = m.astype(F8)
        t8 = jnp.concatenate([xb, xb * xb], axis=1).astype(F8)
        ss_ref[...] = ss_ref[...] + _seg_sum(oh8, t8)
        cnt_ref[...] = cnt_ref[...] + jnp.sum(
            m.astype(F32), axis=1, keepdims=True)

    pltpu.emit_pipeline(
        l0, grid=(nb,),
        in_specs=[pl.BlockSpec((R, C), lambda b: (b, 0))],
        _explicit_indices=True,
    )(f_hbm)

    cnt = jnp.maximum(cnt_ref[...], 1.0)
    mean = ss_ref[:, :C] / cnt
    var = ss_ref[:, C:] / cnt - mean * mean
    rstd = jax.lax.rsqrt(var + EPS)
    meanb = mean.astype(BF)

    def l1(b, _):
        m = onehot_mask(b)
        mu = _gather_rows(m.astype(BF), meanb).astype(BF)
        x = xs_ref[pl.ds(b * R, R), :]
        yb = jnp.maximum(x - mu, jnp.asarray(0.0, BF))
        xs_ref[pl.ds(b * R, R), :] = yb
        s2_ref[...] = s2_ref[...] + _seg_sum(m.astype(F8), yb.astype(F8))
        return 0

    jax.lax.fori_loop(0, nb, l1, 0)

    inst_mean = rstd * s2_ref[...] / cnt
    conv = jnp.dot(inst_mean, T_ref[...], preferred_element_type=F32)
    gate = jax.nn.sigmoid(conv)
    rg = (rstd * gate).astype(BF)

    def l2(idx, out_blk):
        b = idx[0]
        g = _gather_rows(onehot_mask(b).astype(BF), rg)
        out_blk[...] = xs_ref[pl.ds(b * R, R), :].astype(F32) * g

    pltpu.emit_pipeline(
        l2, grid=(nb,),
        out_specs=[pl.BlockSpec((R, C), lambda b: (b, 0))],
        _explicit_indices=True,
    )(out_hbm)


def kernel(features, ins_indices_batch, W1, b1, eca_w):
    N = features.shape[0]
    NB = N // R
    seg3 = ins_indices_batch.reshape(NB, 1, R)
    b1r = b1.reshape(1, C)
    T = (eca_w[0] * jnp.eye(C, k=1) + eca_w[1] * jnp.eye(C)
         + eca_w[2] * jnp.eye(C, k=-1)).astype(F32)

    return pl.pallas_call(
        _outer,
        in_specs=[
            pl.BlockSpec(memory_space=pl.MemorySpace.ANY),
            pl.BlockSpec(memory_space=pltpu.VMEM),
            pl.BlockSpec(memory_space=pltpu.VMEM),
            pl.BlockSpec(memory_space=pltpu.VMEM),
            pl.BlockSpec(memory_space=pltpu.VMEM),
        ],
        out_specs=pl.BlockSpec(memory_space=pl.MemorySpace.ANY),
        out_shape=jax.ShapeDtypeStruct((N, C), F32),
        scratch_shapes=[
            pltpu.VMEM((N, C), BF),
            pltpu.VMEM((I, 2 * C), F32),
            pltpu.VMEM((I, C), F32),
            pltpu.VMEM((I, C), F32),
        ],
    )(features, seg3, W1, b1r, T)


# speedup vs baseline: 1.0295x; 1.0295x over previous
import jax
import jax.numpy as jnp
from jax.experimental import pallas as pl
from jax.experimental.pallas import tpu as pltpu

C = 128
I = 64
EPS = 1e-5
R = 16384
BF = jnp.bfloat16
F8 = jnp.float8_e4m3fn
F32 = jnp.float32


def _gather_rows(oh, ab):
    return jax.lax.dot_general(
        oh, ab.astype(BF), (((0,), (0,)), ((), ())),
        preferred_element_type=F32)


def _seg_sum(oh8, vals8):
    return jax.lax.dot_general(
        oh8, vals8, (((1,), (0,)), ((), ())),
        preferred_element_type=F32)


def _outer(f_hbm, seg_ref, W1_ref, b1_ref, T_ref, out_hbm,
           xs_ref, ss_ref, cnt_ref, s2_ref):
    nb = seg_ref.shape[0]

    def onehot_mask(b):
        seg = seg_ref[b][0, :]
        iota = jax.lax.broadcasted_iota(jnp.int32, (I, R), 0)
        return iota == seg[None, :]

    ss_ref[...] = jnp.zeros_like(ss_ref)
    cnt_ref[...] = jnp.zeros_like(cnt_ref)
    s2_ref[...] = jnp.zeros_like(s2_ref)

    W1b = W1_ref[...].astype(BF)
    b1v = b1_ref[...]

    def l0(idx, f_blk):
        b = idx[0]
        x = jnp.dot(f_blk[...].astype(BF), W1b,
                    preferred_element_type=F32) + b1v
        xb = x.astype(BF)
        xs_ref[pl.ds(b * R, R), :] = xb
        m = onehot_mask(b)
        oh8 = m.astype(F8)
        t8 = jnp.concatenate([xb, xb * xb], axis=1).astype(F8)
        ss_ref[...] = ss_ref[...] + _seg_sum(oh8, t8)
        cnt_ref[...] = cnt_ref[...] + jnp.sum(
            m.astype(F32), axis=1, keepdims=True)

    pltpu.emit_pipeline(
        l0, grid=(nb,),
        in_specs=[pl.BlockSpec((R, C), lambda b: (b, 0))],
        _explicit_indices=True,
    )(f_hbm)

    cnt = jnp.maximum(cnt_ref[...], 1.0)
    mean = ss_ref[:, :C] / cnt
    var = ss_ref[:, C:] / cnt - mean * mean
    rstd = jax.lax.rsqrt(var + EPS)
    meanb = mean.astype(BF)

    def l1(b, _):
        m = onehot_mask(b)
        mu = _gather_rows(m.astype(BF), meanb)
        x = xs_ref[pl.ds(b * R, R), :].astype(F32)
        yb = jnp.maximum(x - mu, 0.0).astype(BF)
        xs_ref[pl.ds(b * R, R), :] = yb
        s2_ref[...] = s2_ref[...] + _seg_sum(m.astype(F8), yb.astype(F8))
        return 0

    jax.lax.fori_loop(0, nb, l1, 0)

    inst_mean = rstd * s2_ref[...] / cnt
    conv = jnp.dot(inst_mean, T_ref[...], preferred_element_type=F32)
    gate = jax.nn.sigmoid(conv)
    rg = (rstd * gate).astype(BF)

    def l2(idx, out_blk):
        b = idx[0]
        g = _gather_rows(onehot_mask(b).astype(BF), rg)
        out_blk[...] = xs_ref[pl.ds(b * R, R), :].astype(F32) * g

    pltpu.emit_pipeline(
        l2, grid=(nb,),
        out_specs=[pl.BlockSpec((R, C), lambda b: (b, 0))],
        _explicit_indices=True,
    )(out_hbm)


def kernel(features, ins_indices_batch, W1, b1, eca_w):
    N = features.shape[0]
    NB = N // R
    seg3 = ins_indices_batch.reshape(NB, 1, R)
    b1r = b1.reshape(1, C)
    T = (eca_w[0] * jnp.eye(C, k=1) + eca_w[1] * jnp.eye(C)
         + eca_w[2] * jnp.eye(C, k=-1)).astype(F32)

    return pl.pallas_call(
        _outer,
        in_specs=[
            pl.BlockSpec(memory_space=pl.MemorySpace.ANY),
            pl.BlockSpec(memory_space=pltpu.VMEM),
            pl.BlockSpec(memory_space=pltpu.VMEM),
            pl.BlockSpec(memory_space=pltpu.VMEM),
            pl.BlockSpec(memory_space=pltpu.VMEM),
        ],
        out_specs=pl.BlockSpec(memory_space=pl.MemorySpace.ANY),
        out_shape=jax.ShapeDtypeStruct((N, C), F32),
        scratch_shapes=[
            pltpu.VMEM((N, C), BF),
            pltpu.VMEM((I, 2 * C), F32),
            pltpu.VMEM((I, C), F32),
            pltpu.VMEM((I, C), F32),
        ],
    )(features, seg3, W1, b1r, T)
